# speedup vs baseline: 1.1364x; 1.1364x over previous
import functools
import jax
import jax.numpy as jnp
from jax.experimental import pallas as pl
from jax.experimental.pallas import tpu as pltpu

N = 10000
E = 320000
F_IN = 128
HID = 64
HEADS = 8
HID2 = 32
NUM_CLASS = 40

BN_SCALE = 1.0 / (1.0 + 1e-5) ** 0.5


def _lrelu(x, s):
    return jnp.maximum(x, s * x)


def _tail_body(y2_ref, w_ref, b_ref, o_ref):
    y = y2_ref[...]
    logits = jax.lax.dot_general(y, w_ref[...], (((1,), (1,)), ((), ())),
                                 preferred_element_type=jnp.float32)
    logits = logits + b_ref[...]
    m = jnp.max(logits, axis=-1, keepdims=True)
    ex = jnp.exp(logits - m)
    lse = jnp.log(jnp.sum(ex, axis=-1, keepdims=True)) + m
    o_ref[...] = logits - lse


def _tail(y2, W2, b2):
    rows = 1000
    return pl.pallas_call(
        _tail_body,
        grid=(N // rows,),
        in_specs=[
            pl.BlockSpec((rows, HID2), lambda i: (i, 0)),
            pl.BlockSpec((NUM_CLASS, HID2), lambda i: (0, 0)),
            pl.BlockSpec((NUM_CLASS,), lambda i: (0,)),
        ],
        out_specs=pl.BlockSpec((rows, NUM_CLASS), lambda i: (i, 0)),
        out_shape=jax.ShapeDtypeStruct((N, NUM_CLASS), jnp.float32),
    )(y2, W2, b2)


def _gat_simple(x, src, dst, W, att_src, att_dst, bias, heads, out_ch):
    n = x.shape[0]
    h = (x @ W.T).reshape(n, heads, out_ch)
    a_src = jnp.sum(h * att_src, axis=-1)
    a_dst = jnp.sum(h * att_dst, axis=-1)
    ex = jnp.exp(_lrelu(a_src[src] + a_dst[dst], 0.2))
    den_e = jax.ops.segment_sum(ex, dst, num_segments=n)
    es = jnp.exp(_lrelu(a_src + a_dst, 0.2))
    rden = 1.0 / (den_e + es)
    coef = ex * rden[dst]
    out = jax.ops.segment_sum(h[src] * coef[:, :, None], dst, num_segments=n)
    out = out + h * (es * rden)[:, :, None]
    return out.reshape(n, heads * out_ch) + bias


def kernel(x, edge_index, edge_weight, W1, b1, Wc1, as1, ad1, bc1, Wc2, as2,
           ad2, bc2, g1, be1, g2, be2, W2, b2):
    src = edge_index[0]
    dst = edge_index[1]
    h0 = _lrelu(x @ W1.T + b1, 0.01)
    y1 = _lrelu(_gat_simple(h0, src, dst, Wc1, as1, ad1, bc1, HEADS, HID), 0.01)
    y1 = y1 * BN_SCALE * g1 + be1
    y2 = _lrelu(_gat_simple(y1, src, dst, Wc2, as2, ad2, bc2, 1, HID2), 0.01)
    y2 = y2 * BN_SCALE * g2 + be2
    return _tail(y2, W2, b2)


# speedup vs baseline: 4.8870x; 4.3006x over previous
import functools
import jax
import jax.numpy as jnp
from jax import lax
from jax.experimental import pallas as pl
from jax.experimental.pallas import tpu as pltpu
from jax.experimental.pallas import tpu_sc as plsc

N = 10000
E = 320000
F_IN = 128
HID = 64
HEADS = 8
HID2 = 32
NUM_CLASS = 40

BN_SCALE = 1.0 / (1.0 + 1e-5) ** 0.5

_NTILE = 16
_K = 80
_NP = 10240
_GPT1 = E // (_NTILE * _K)
_GPT2 = E // (2 * _NTILE * _K)
_RPT = _NP // _NTILE
_CH = 10
_NCH = _GPT1 // _CH
_CH2 = 5
_NCH2 = _GPT2 // _CH2
_NP4 = _NP // 4
_RPT4 = _NP4 // _NTILE

_mesh = plsc.VectorSubcoreMesh(core_axis_name="c", subcore_axis_name="s")


def _lrelu(x, s):
    return jnp.maximum(x, s * x)


def _bcast_lane(v, j):
    return lax.gather(
        v, jnp.full((16, 1), j, jnp.int32),
        dimension_numbers=lax.GatherDimensionNumbers(
            offset_dims=(), collapsed_slice_dims=(0,), start_index_map=(0,)),
        slice_sizes=(1,), mode=lax.GatherScatterMode.PROMISE_IN_BOUNDS)


@functools.partial(
    pl.kernel,
    out_type=jax.ShapeDtypeStruct((4 * _NP, 128), jnp.float32),
    mesh=_mesh,
    scratch_types=[
        pltpu.VMEM((_CH, _K), jnp.int32),
        pltpu.VMEM((_CH, _K), jnp.int32),
        pltpu.VMEM((_CH, _K), jnp.float32),
        pltpu.VMEM((_CH, _K), jnp.float32),
        pltpu.VMEM((_K, 128), jnp.float32),
        pltpu.VMEM_SHARED((_NP, 128), jnp.float32),
        pltpu.SemaphoreType.DMA,
    ],
)
def _mp1(hcat, selfcat, src4, dst4, coefT, out,
         srcb, dstb, c0b, c1b, rows, acc, sem):
    core = lax.axis_index("c")
    sub = lax.axis_index("s")
    for p in range(2):
        chunk = 2 * core + p
        roff = chunk * _NP

        pltpu.sync_copy(selfcat.at[pl.ds(roff + sub * _RPT, _RPT)],
                        acc.at[pl.ds(sub * _RPT, _RPT)])
        plsc.subcore_barrier()

        def _chunk(ch, carry):
            pltpu.sync_copy(src4.at[sub, ch], srcb)
            pltpu.sync_copy(dst4.at[sub, ch], dstb)
            pltpu.sync_copy(coefT.at[2 * chunk, sub, ch], c0b)
            pltpu.sync_copy(coefT.at[2 * chunk + 1, sub, ch], c1b)

            def _adj(i, c2):
                for q in range(_K // 16):
                    srcb[i, pl.ds(q * 16, 16)] = (
                        srcb[i, pl.ds(q * 16, 16)] + roff)
                return c2
            lax.fori_loop(0, _CH, _adj, 0)

            def _group(g, c2):
                pltpu.async_copy(hcat.at[srcb.at[g]], rows, sem).wait()
                for s16 in range(_K // 16):
                    c0v = c0b[g, pl.ds(s16 * 16, 16)]
                    c1v = c1b[g, pl.ds(s16 * 16, 16)]
                    for jj in range(16):
                        j = s16 * 16 + jj
                        b0 = _bcast_lane(c0v, jj)
                        b1 = _bcast_lane(c1v, jj)
                        for q in range(4):
                            rows[j, pl.ds(q * 16, 16)] = (
                                rows[j, pl.ds(q * 16, 16)] * b0)
                        for q in range(4, 8):
                            rows[j, pl.ds(q * 16, 16)] = (
                                rows[j, pl.ds(q * 16, 16)] * b1)
                pltpu.sync_copy(rows, acc.at[dstb.at[g]], add=True)
                return c2
            lax.fori_loop(0, _CH, _group, 0)
            return carry
        lax.fori_loop(0, _NCH, _chunk, 0)
        plsc.subcore_barrier()
        pltpu.sync_copy(acc.at[pl.ds(sub * _RPT, _RPT)],
                        out.at[pl.ds(roff + sub * _RPT, _RPT)])
        plsc.subcore_barrier()


@functools.partial(
    pl.kernel,
    out_type=jax.ShapeDtypeStruct((2, _NP4, 128), jnp.float32),
    mesh=_mesh,
    scratch_types=[
        pltpu.VMEM((_CH2, _K), jnp.int32),
        pltpu.VMEM((_CH2, _K), jnp.int32),
        pltpu.VMEM((_CH2, _K), jnp.float32),
        pltpu.VMEM((_K, 128), jnp.float32),
        pltpu.VMEM_SHARED((_NP4, 128), jnp.float32),
        pltpu.SemaphoreType.DMA,
    ],
)
def _mp2(h2p4, init2, src4, dst4, coef4, out,
         srcb, dstb, c2b, rows, acc, sem):
    core = lax.axis_index("c")
    sub = lax.axis_index("s")
    wid = core * _NTILE + sub
    pltpu.sync_copy(init2.at[core, pl.ds(sub * _RPT4, _RPT4)],
                    acc.at[pl.ds(sub * _RPT4, _RPT4)])
    plsc.subcore_barrier()

    def _chunk(ch, carry):
        pltpu.sync_copy(src4.at[wid, ch], srcb)
        pltpu.sync_copy(dst4.at[wid, ch], dstb)
        pltpu.sync_copy(coef4.at[wid, ch], c2b)

        def _group(g, c2):
            pltpu.async_copy(h2p4.at[srcb.at[g]], rows, sem).wait()
            for s16 in range(_K // 16):
                cv = c2b[g, pl.ds(s16 * 16, 16)]
                for jj in range(16):
                    j = s16 * 16 + jj
                    b = _bcast_lane(cv, jj)
                    for q in range(8):
                        rows[j, pl.ds(q * 16, 16)] = (
                            rows[j, pl.ds(q * 16, 16)] * b)
            pltpu.sync_copy(rows, acc.at[dstb.at[g]], add=True)
            return c2
        lax.fori_loop(0, _CH2, _group, 0)
        return carry
    lax.fori_loop(0, _NCH2, _chunk, 0)
    plsc.subcore_barrier()
    pltpu.sync_copy(acc.at[pl.ds(sub * _RPT4, _RPT4)],
                    out.at[core, pl.ds(sub * _RPT4, _RPT4)])


def _tail_body(y2_ref, w_ref, b_ref, o_ref):
    y = y2_ref[...]
    logits = lax.dot_general(y, w_ref[...], (((1,), (1,)), ((), ())),
                             preferred_element_type=jnp.float32)
    logits = logits + b_ref[...]
    m = jnp.max(logits, axis=-1, keepdims=True)
    ex = jnp.exp(logits - m)
    lse = jnp.log(jnp.sum(ex, axis=-1, keepdims=True)) + m
    o_ref[...] = logits - lse


def _tail(y2, W2, b2):
    rows = 1000
    return pl.pallas_call(
        _tail_body,
        grid=(N // rows,),
        in_specs=[
            pl.BlockSpec((rows, HID2), lambda i: (i, 0)),
            pl.BlockSpec((NUM_CLASS, HID2), lambda i: (0, 0)),
            pl.BlockSpec((NUM_CLASS,), lambda i: (0,)),
        ],
        out_specs=pl.BlockSpec((rows, NUM_CLASS), lambda i: (i, 0)),
        out_shape=jax.ShapeDtypeStruct((N, NUM_CLASS), jnp.float32),
    )(y2, W2, b2)


def kernel(x, edge_index, edge_weight, W1, b1, Wc1, as1, ad1, bc1, Wc2, as2,
           ad2, bc2, g1, be1, g2, be2, W2, b2):
    src = edge_index[0]
    dst = edge_index[1]
    src3a = src.reshape(_NTILE, _NCH, _CH, _K)
    dst3a = dst.reshape(_NTILE, _NCH, _CH, _K)
    shp2 = (2 * _NTILE, _NCH2, _CH2, _K)
    src4b = (4 * src + dst % 4).reshape(shp2)
    dst4b = (dst // 4).reshape(shp2)

    def _padn(a):
        return jnp.pad(a, ((0, _NP - N),) + ((0, 0),) * (a.ndim - 1))

    h0 = _lrelu(x @ W1.T + b1, 0.01)
    h1 = (h0 @ Wc1.T).reshape(N, HEADS, HID)
    a_src = jnp.sum(h1 * as1, axis=-1)
    a_dst = jnp.sum(h1 * ad1, axis=-1)
    ex = jnp.exp(_lrelu(a_src[src] + a_dst[dst], 0.2))
    den_e = jax.ops.segment_sum(ex, dst, num_segments=N)
    es = jnp.exp(_lrelu(a_src + a_dst, 0.2))
    rden = 1.0 / (den_e + es)
    coefT = (ex * rden[dst]).T.reshape(HEADS, _NTILE, _NCH, _CH, _K)

    hcat = jnp.transpose(
        _padn(h1.reshape(N, 4, 128)), (1, 0, 2)).reshape(4 * _NP, 128)
    selfc = h1 * (es * rden)[:, :, None]
    selfcat = jnp.transpose(
        _padn(selfc.reshape(N, 4, 128)), (1, 0, 2)).reshape(4 * _NP, 128)

    out1 = _mp1(hcat, selfcat, src3a, dst3a, coefT)
    out1 = jnp.transpose(out1.reshape(4, _NP, 128), (1, 0, 2))[:N].reshape(
        N, HEADS * HID)

    y1 = _lrelu(out1 + bc1, 0.01) * (BN_SCALE * g1) + be1

    h2 = y1 @ Wc2.T
    a_src2 = jnp.sum(h2 * as2[0], axis=-1)
    a_dst2 = jnp.sum(h2 * ad2[0], axis=-1)
    ex2 = jnp.exp(_lrelu(a_src2[src] + a_dst2[dst], 0.2))
    den2 = jax.ops.segment_sum(ex2, dst, num_segments=N)
    es2 = jnp.exp(_lrelu(a_src2 + a_dst2, 0.2))
    rden2 = 1.0 / (den2 + es2)
    coef4 = (ex2 * rden2[dst]).reshape(shp2)
    h2pn = _padn(h2)
    h2p4 = jnp.stack(
        [jnp.pad(h2pn, ((0, 0), (32 * t, 96 - 32 * t))) for t in range(4)],
        axis=1).reshape(4 * _NP, 128)
    init2 = jnp.stack([
        _padn(h2 * (es2 * rden2)[:, None]).reshape(_NP4, 128),
        jnp.zeros((_NP4, 128), jnp.float32)])

    out2p = _mp2(h2p4, init2, src4b, dst4b, coef4)
    out2 = (out2p[0] + out2p[1]).reshape(_NP, HID2)[:N]
    y2 = _lrelu(out2 + bc2, 0.01) * (BN_SCALE * g2) + be2

    return _tail(y2, W2, b2)


# speedup vs baseline: 16.7569x; 3.4289x over previous
import functools
import jax
import jax.numpy as jnp
from jax import lax
from jax.experimental import pallas as pl
from jax.experimental.pallas import tpu as pltpu
from jax.experimental.pallas import tpu_sc as plsc

N = 10000
E = 320000
F_IN = 128
HID = 64
HEADS = 8
HID2 = 32
NUM_CLASS = 40

BN_SCALE = 1.0 / (1.0 + 1e-5) ** 0.5

_NTILE = 16
_K = 80
_NP = 10240
_GPT1 = E // (_NTILE * _K)
_GPT2 = E // (2 * _NTILE * _K)
_RPT = _NP // _NTILE
_CH = 5
_NCH = _GPT1 // _CH
_NP4 = _NP // 4
_KP = 80
_CHP = 10
_NCHP = E // (4 * _CHP * _KP)

_mesh = plsc.VectorSubcoreMesh(core_axis_name="c", subcore_axis_name="s")


def _lrelu(x, s):
    return jnp.maximum(x, s * x)


def _bcast_lane(v, j):
    return lax.gather(
        v, jnp.full((16, 1), j, jnp.int32),
        dimension_numbers=lax.GatherDimensionNumbers(
            offset_dims=(), collapsed_slice_dims=(0,), start_index_map=(0,)),
        slice_sizes=(1,), mode=lax.GatherScatterMode.PROMISE_IN_BOUNDS)


@functools.partial(
    pl.kernel,
    out_type=jax.ShapeDtypeStruct((4 * _NP, 128), jnp.float32),
    mesh=_mesh,
    scratch_types=[
        pltpu.VMEM((_CH, _K), jnp.int32),
        pltpu.VMEM((_CH, _K), jnp.int32),
        pltpu.VMEM((_CH, _K), jnp.float32),
        pltpu.VMEM((_CH, _K), jnp.float32),
        pltpu.VMEM((_K, 128), jnp.float32),
        pltpu.VMEM_SHARED((_NP, 128), jnp.float32),
        pltpu.SemaphoreType.DMA,
    ],
)
def _mp1(hcat, selfcat, src4, dst4, coefT, out,
         srcb, dstb, c0b, c1b, rows, acc, sem):
    core = lax.axis_index("c")
    sub = lax.axis_index("s")
    for p in range(2):
        chunk = 2 * core + p
        roff = chunk * _NP

        pltpu.sync_copy(selfcat.at[pl.ds(roff + sub * _RPT, _RPT)],
                        acc.at[pl.ds(sub * _RPT, _RPT)])
        plsc.subcore_barrier()

        def _chunk(ch, carry):
            pltpu.sync_copy(src4.at[sub, ch], srcb)
            pltpu.sync_copy(dst4.at[sub, ch], dstb)
            pltpu.sync_copy(coefT.at[2 * chunk, sub, ch], c0b)
            pltpu.sync_copy(coefT.at[2 * chunk + 1, sub, ch], c1b)

            def _adj(i, c2):
                for q in range(_K // 16):
                    srcb[i, pl.ds(q * 16, 16)] = (
                        srcb[i, pl.ds(q * 16, 16)] + roff)
                return c2
            lax.fori_loop(0, _CH, _adj, 0)

            def _group(g, c2):
                pltpu.async_copy(hcat.at[srcb.at[g]], rows, sem).wait()
                for s16 in range(_K // 16):
                    c0v = c0b[g, pl.ds(s16 * 16, 16)]
                    c1v = c1b[g, pl.ds(s16 * 16, 16)]
                    for jj in range(16):
                        j = s16 * 16 + jj
                        b0 = _bcast_lane(c0v, jj)
                        b1 = _bcast_lane(c1v, jj)
                        for q in range(4):
                            rows[j, pl.ds(q * 16, 16)] = (
                                rows[j, pl.ds(q * 16, 16)] * b0)
                        for q in range(4, 8):
                            rows[j, pl.ds(q * 16, 16)] = (
                                rows[j, pl.ds(q * 16, 16)] * b1)
                pltpu.sync_copy(rows, acc.at[dstb.at[g]], add=True)
                return c2
            lax.fori_loop(0, _CH, _group, 0)
            return carry
        lax.fori_loop(0, _NCH, _chunk, 0)
        plsc.subcore_barrier()
        pltpu.sync_copy(acc.at[pl.ds(sub * _RPT, _RPT)],
                        out.at[pl.ds(roff + sub * _RPT, _RPT)])
        plsc.subcore_barrier()


@functools.partial(
    pl.kernel,
    out_type=(jax.ShapeDtypeStruct((HEADS, 4, _NCHP, _CHP, _KP), jnp.float32),
              jax.ShapeDtypeStruct((32 * _NP,), jnp.float32)),
    mesh=_mesh,
    compiler_params=pltpu.CompilerParams(needs_layout_passes=False),
    scratch_types=[
        pltpu.VMEM((_NP,), jnp.float32),
        pltpu.VMEM((_NP,), jnp.float32),
        pltpu.VMEM((_NP,), jnp.float32),
        pltpu.VMEM((_CHP, _KP), jnp.int32),
        pltpu.VMEM((_CHP, _KP), jnp.int32),
        pltpu.VMEM((_CHP, _KP), jnp.float32),
        pltpu.SemaphoreType.DMA,
    ],
)
def _att(aT, bT, srcP, dstP, exT, denP,
         asr, adr, den, srcb, dstb, exb, sem):
    core = lax.axis_index("c")
    sub = lax.axis_index("s")
    hl = sub // 4
    q = sub % 4
    h = core * 4 + hl
    w = core * 16 + sub
    zeros16 = jnp.zeros((16,), jnp.float32)

    pltpu.sync_copy(aT.at[pl.ds(h * _NP, _NP)], asr)
    pltpu.sync_copy(bT.at[pl.ds(h * _NP, _NP)], adr)

    def _zrow(i, c2):
        den[pl.ds(i * 16, 16)] = zeros16
        return c2
    lax.fori_loop(0, _NP // 16, _zrow, 0)

    def _chunk1(ch, carry):
        pltpu.sync_copy(srcP.at[q, ch], srcb)
        pltpu.sync_copy(dstP.at[q, ch], dstb)

        def _group(g, c2):
            for t in range(_KP // 16):
                sv = srcb[g, pl.ds(t * 16, 16)]
                dv = dstb[g, pl.ds(t * 16, 16)]
                av = plsc.load_gather(asr, [sv])
                bv = plsc.load_gather(adr, [dv])
                al = av + bv
                al = jnp.maximum(al, 0.2 * al)
                e = jnp.exp(al)
                exb[g, pl.ds(t * 16, 16)] = e
                plsc.addupdate_scatter(den, [dv], e)
            return c2
        lax.fori_loop(0, _CHP, _group, 0)
        pltpu.sync_copy(exb, exT.at[h, q, ch])
        return carry
    lax.fori_loop(0, _NCHP, _chunk1, 0)
    pltpu.sync_copy(den, denP.at[pl.ds(w * _NP, _NP)])


def _tail_body(y2_ref, w_ref, b_ref, o_ref):
    y = y2_ref[...]
    logits = lax.dot_general(y, w_ref[...], (((1,), (1,)), ((), ())),
                             preferred_element_type=jnp.float32)
    logits = logits + b_ref[...]
    m = jnp.max(logits, axis=-1, keepdims=True)
    ex = jnp.exp(logits - m)
    lse = jnp.log(jnp.sum(ex, axis=-1, keepdims=True)) + m
    o_ref[...] = logits - lse


def _tail(y2, W2, b2):
    rows = 1000
    return pl.pallas_call(
        _tail_body,
        grid=(N // rows,),
        in_specs=[
            pl.BlockSpec((rows, HID2), lambda i: (i, 0)),
            pl.BlockSpec((NUM_CLASS, HID2), lambda i: (0, 0)),
            pl.BlockSpec((NUM_CLASS,), lambda i: (0,)),
        ],
        out_specs=pl.BlockSpec((rows, NUM_CLASS), lambda i: (i, 0)),
        out_shape=jax.ShapeDtypeStruct((N, NUM_CLASS), jnp.float32),
    )(y2, W2, b2)


def kernel(x, edge_index, edge_weight, W1, b1, Wc1, as1, ad1, bc1, Wc2, as2,
           ad2, bc2, g1, be1, g2, be2, W2, b2):
    src = edge_index[0]
    dst = edge_index[1]
    shp1 = (_NTILE, _NCH, _CH, _K)
    src3a = src.reshape(shp1)
    dst3a = dst.reshape(shp1)
    dst3a4 = (dst // 4).reshape(shp1)
    srcP = src.reshape(4, _NCHP, _CHP, _KP)
    dstP = dst.reshape(4, _NCHP, _CHP, _KP)

    def _padn(a):
        return jnp.pad(a, ((0, _NP - N),) + ((0, 0),) * (a.ndim - 1))

    h0 = _lrelu(x @ W1.T + b1, 0.01)
    h1 = (h0 @ Wc1.T).reshape(N, HEADS, HID)
    a_src = jnp.sum(h1 * as1, axis=-1)
    a_dst = jnp.sum(h1 * ad1, axis=-1)
    aT1 = _padn(a_src).T.reshape(HEADS * _NP)
    bT1 = _padn(a_dst).T.reshape(HEADS * _NP)

    exP1, denP1 = _att(aT1, bT1, srcP, dstP)
    den1 = denP1.reshape(2, 4, 4, _NP).sum(2).reshape(HEADS, _NP)[:, :N].T
    es1 = jnp.exp(_lrelu(a_src + a_dst, 0.2))
    rden1 = 1.0 / (den1 + es1)
    coefT1 = exP1.reshape(HEADS, E).reshape((HEADS,) + shp1)

    hcat = jnp.transpose(
        _padn(h1.reshape(N, 4, 128)), (1, 0, 2)).reshape(4 * _NP, 128)
    selfc = h1 * es1[:, :, None]
    selfcat = jnp.transpose(
        _padn(selfc.reshape(N, 4, 128)), (1, 0, 2)).reshape(4 * _NP, 128)

    out1 = _mp1(hcat, selfcat, src3a, dst3a, coefT1)
    out1 = jnp.transpose(out1.reshape(4, _NP, 128), (1, 0, 2))[:N].reshape(
        N, HEADS, HID)
    out1 = (out1 * rden1[:, :, None]).reshape(N, HEADS * HID)
    y1 = _lrelu(out1 + bc1, 0.01) * (BN_SCALE * g1) + be1

    h2 = y1 @ Wc2.T
    a_src2 = jnp.sum(h2 * as2[0], axis=-1)
    a_dst2 = jnp.sum(h2 * ad2[0], axis=-1)
    z7 = jnp.zeros((7 * _NP,), jnp.float32)
    aT2 = jnp.concatenate([_padn(a_src2), z7])
    bT2 = jnp.concatenate([_padn(a_dst2), z7])

    exP2, denP2 = _att(aT2, bT2, srcP, dstP)
    den2 = (denP2[0:_NP] + denP2[_NP:2 * _NP] + denP2[2 * _NP:3 * _NP]
            + denP2[3 * _NP:4 * _NP])[:N]
    es2 = jnp.exp(_lrelu(a_src2 + a_dst2, 0.2))
    rden2 = 1.0 / (den2 + es2)
    ex2 = exP2[0].reshape(E)

    dm = dst % 4
    z = jnp.zeros((E,), jnp.float32)
    m = [jnp.where(dm == c, ex2, 0.0) for c in range(4)]
    coefT2 = jnp.stack([m[0], z, m[1], z, z, m[2], z, m[3]]).reshape(
        (HEADS,) + shp1)

    h2pn = _padn(h2)
    h2cat = jnp.concatenate(
        [jnp.pad(h2pn, ((0, 0), (32 * c, 96 - 32 * c))) for c in range(4)],
        axis=0)
    self2 = _padn(h2 * es2[:, None]).reshape(_NP4, 128)
    selfcat2 = jnp.concatenate(
        [self2, jnp.zeros((4 * _NP - _NP4, 128), jnp.float32)], axis=0)

    out2p = _mp1(h2cat, selfcat2, src3a, dst3a4, coefT2)
    out2 = (out2p[0 * _NP:0 * _NP + _NP4] + out2p[1 * _NP:1 * _NP + _NP4]
            + out2p[2 * _NP:2 * _NP + _NP4] + out2p[3 * _NP:3 * _NP + _NP4])
    out2 = out2.reshape(_NP, HID2)[:N] * rden2[:, None]
    y2 = _lrelu(out2 + bc2, 0.01) * (BN_SCALE * g2) + be2

    return _tail(y2, W2, b2)


# speedup vs baseline: 25.8650x; 1.5435x over previous
import functools
import jax
import jax.numpy as jnp
from jax import lax
from jax.experimental import pallas as pl
from jax.experimental.pallas import tpu as pltpu
from jax.experimental.pallas import tpu_sc as plsc

N = 10000
E = 320000
F_IN = 128
HID = 64
HEADS = 8
HID2 = 32
NUM_CLASS = 40

BN_SCALE = 1.0 / (1.0 + 1e-5) ** 0.5

_NTILE = 16
_K = 80
_NP = 10240
_GPT1 = E // (_NTILE * _K)
_GPT2 = E // (2 * _NTILE * _K)
_RPT = _NP // _NTILE
_CH = 5
_NCH = _GPT1 // _CH
_NP4 = _NP // 4
_KP = 80
_CHP = 10
_NCHP = E // (4 * _CHP * _KP)

_mesh = plsc.VectorSubcoreMesh(core_axis_name="c", subcore_axis_name="s")


def _lrelu(x, s):
    return jnp.maximum(x, s * x)


def _bcast_lane(v, j):
    return lax.gather(
        v, jnp.full((16, 1), j, jnp.int32),
        dimension_numbers=lax.GatherDimensionNumbers(
            offset_dims=(), collapsed_slice_dims=(0,), start_index_map=(0,)),
        slice_sizes=(1,), mode=lax.GatherScatterMode.PROMISE_IN_BOUNDS)


@functools.partial(
    pl.kernel,
    out_type=jax.ShapeDtypeStruct((4 * _NP, 128), jnp.float32),
    mesh=_mesh,
    scratch_types=[
        pltpu.VMEM((_CH, _K), jnp.int32),
        pltpu.VMEM((_CH, _K), jnp.int32),
        pltpu.VMEM((_CH, _K), jnp.float32),
        pltpu.VMEM((_CH, _K), jnp.float32),
        pltpu.VMEM((_K, 128), jnp.float32),
        pltpu.VMEM_SHARED((_NP, 128), jnp.float32),
        pltpu.SemaphoreType.DMA,
    ],
)
def _mp1(hcat, selfcat, src4, dst4, coefT, out,
         srcb, dstb, c0b, c1b, rows, acc, sem):
    core = lax.axis_index("c")
    sub = lax.axis_index("s")
    for p in range(2):
        chunk = 2 * core + p
        roff = chunk * _NP

        pltpu.sync_copy(selfcat.at[pl.ds(roff + sub * _RPT, _RPT)],
                        acc.at[pl.ds(sub * _RPT, _RPT)])
        plsc.subcore_barrier()

        def _chunk(ch, carry):
            pltpu.sync_copy(src4.at[sub, ch], srcb)
            pltpu.sync_copy(dst4.at[sub, ch], dstb)
            pltpu.sync_copy(coefT.at[2 * chunk, sub, ch], c0b)
            pltpu.sync_copy(coefT.at[2 * chunk + 1, sub, ch], c1b)

            def _adj(i, c2):
                for q in range(_K // 16):
                    srcb[i, pl.ds(q * 16, 16)] = (
                        srcb[i, pl.ds(q * 16, 16)] + roff)
                return c2
            lax.fori_loop(0, _CH, _adj, 0)

            def _group(g, c2):
                pltpu.async_copy(hcat.at[srcb.at[g]], rows, sem).wait()
                for s16 in range(_K // 16):
                    c0v = c0b[g, pl.ds(s16 * 16, 16)]
                    c1v = c1b[g, pl.ds(s16 * 16, 16)]
                    for jj in range(16):
                        j = s16 * 16 + jj
                        b0 = _bcast_lane(c0v, jj)
                        b1 = _bcast_lane(c1v, jj)
                        for q in range(4):
                            rows[j, pl.ds(q * 16, 16)] = (
                                rows[j, pl.ds(q * 16, 16)] * b0)
                        for q in range(4, 8):
                            rows[j, pl.ds(q * 16, 16)] = (
                                rows[j, pl.ds(q * 16, 16)] * b1)
                pltpu.sync_copy(rows, acc.at[dstb.at[g]], add=True)
                return c2
            lax.fori_loop(0, _CH, _group, 0)
            return carry
        lax.fori_loop(0, _NCH, _chunk, 0)
        plsc.subcore_barrier()
        pltpu.sync_copy(acc.at[pl.ds(sub * _RPT, _RPT)],
                        out.at[pl.ds(roff + sub * _RPT, _RPT)])
        plsc.subcore_barrier()


@functools.partial(
    pl.kernel,
    out_type=jax.ShapeDtypeStruct((2 * _NP, 32), jnp.float32),
    mesh=_mesh,
    scratch_types=[
        pltpu.VMEM((5, _K), jnp.int32),
        pltpu.VMEM((5, _K), jnp.int32),
        pltpu.VMEM((5, _K), jnp.float32),
        pltpu.VMEM((_K, 32), jnp.float32),
        pltpu.VMEM_SHARED((_NP, 32), jnp.float32),
        pltpu.VMEM_SHARED((_NP, 32), jnp.float32),
        pltpu.SemaphoreType.DMA,
    ],
)
def _mp2n(h2tab, init2, src32, dst32, coef32, out,
          srcb, dstb, cb, rows, acc, h2s, sem):
    core = lax.axis_index("c")
    sub = lax.axis_index("s")
    w = core * _NTILE + sub
    pltpu.sync_copy(init2.at[pl.ds(core * _NP + sub * _RPT, _RPT)],
                    acc.at[pl.ds(sub * _RPT, _RPT)])
    pltpu.sync_copy(h2tab.at[pl.ds(sub * _RPT, _RPT)],
                    h2s.at[pl.ds(sub * _RPT, _RPT)])
    plsc.subcore_barrier()

    def _chunk(ch, carry):
        pltpu.sync_copy(src32.at[w, ch], srcb)
        pltpu.sync_copy(dst32.at[w, ch], dstb)
        pltpu.sync_copy(coef32.at[w, ch], cb)

        def _group(g, c2):
            pltpu.async_copy(h2s.at[srcb.at[g]], rows, sem).wait()
            for s16 in range(_K // 16):
                cv = cb[g, pl.ds(s16 * 16, 16)]
                for jj in range(16):
                    j = s16 * 16 + jj
                    b = _bcast_lane(cv, jj)
                    rows[j, pl.ds(0, 16)] = rows[j, pl.ds(0, 16)] * b
                    rows[j, pl.ds(16, 16)] = rows[j, pl.ds(16, 16)] * b
            pltpu.sync_copy(rows, acc.at[dstb.at[g]], add=True)
            return c2
        lax.fori_loop(0, 5, _group, 0)
        return carry
    lax.fori_loop(0, 25, _chunk, 0)
    plsc.subcore_barrier()
    pltpu.sync_copy(acc.at[pl.ds(sub * _RPT, _RPT)],
                    out.at[pl.ds(core * _NP + sub * _RPT, _RPT)])


@functools.partial(
    pl.kernel,
    out_type=(jax.ShapeDtypeStruct((HEADS, 4, _NCHP, _CHP, _KP), jnp.float32),
              jax.ShapeDtypeStruct((32 * _NP,), jnp.float32)),
    mesh=_mesh,
    compiler_params=pltpu.CompilerParams(needs_layout_passes=False),
    scratch_types=[
        pltpu.VMEM((_NP,), jnp.float32),
        pltpu.VMEM((_NP,), jnp.float32),
        pltpu.VMEM((_NP,), jnp.float32),
        pltpu.VMEM((_CHP, _KP), jnp.int32),
        pltpu.VMEM((_CHP, _KP), jnp.int32),
        pltpu.VMEM((_CHP, _KP), jnp.float32),
        pltpu.SemaphoreType.DMA,
    ],
)
def _att(aT, bT, srcP, dstP, exT, denP,
         asr, adr, den, srcb, dstb, exb, sem):
    core = lax.axis_index("c")
    sub = lax.axis_index("s")
    hl = sub // 4
    q = sub % 4
    h = core * 4 + hl
    w = core * 16 + sub
    zeros16 = jnp.zeros((16,), jnp.float32)

    pltpu.sync_copy(aT.at[pl.ds(h * _NP, _NP)], asr)
    pltpu.sync_copy(bT.at[pl.ds(h * _NP, _NP)], adr)

    def _zrow(i, c2):
        den[pl.ds(i * 16, 16)] = zeros16
        return c2
    lax.fori_loop(0, _NP // 16, _zrow, 0)

    def _chunk1(ch, carry):
        pltpu.sync_copy(srcP.at[q, ch], srcb)
        pltpu.sync_copy(dstP.at[q, ch], dstb)

        def _group(g, c2):
            for t in range(_KP // 16):
                sv = srcb[g, pl.ds(t * 16, 16)]
                dv = dstb[g, pl.ds(t * 16, 16)]
                av = plsc.load_gather(asr, [sv])
                bv = plsc.load_gather(adr, [dv])
                al = av + bv
                al = jnp.maximum(al, 0.2 * al)
                e = jnp.exp(al)
                exb[g, pl.ds(t * 16, 16)] = e
                plsc.addupdate_scatter(den, [dv], e)
            return c2
        lax.fori_loop(0, _CHP, _group, 0)
        pltpu.sync_copy(exb, exT.at[h, q, ch])
        return carry
    lax.fori_loop(0, _NCHP, _chunk1, 0)
    pltpu.sync_copy(den, denP.at[pl.ds(w * _NP, _NP)])


def _tail_body(y2_ref, w_ref, b_ref, o_ref):
    y = y2_ref[...]
    logits = lax.dot_general(y, w_ref[...], (((1,), (1,)), ((), ())),
                             preferred_element_type=jnp.float32)
    logits = logits + b_ref[...]
    m = jnp.max(logits, axis=-1, keepdims=True)
    ex = jnp.exp(logits - m)
    lse = jnp.log(jnp.sum(ex, axis=-1, keepdims=True)) + m
    o_ref[...] = logits - lse


def _tail(y2, W2, b2):
    rows = 1000
    return pl.pallas_call(
        _tail_body,
        grid=(N // rows,),
        in_specs=[
            pl.BlockSpec((rows, HID2), lambda i: (i, 0)),
            pl.BlockSpec((NUM_CLASS, HID2), lambda i: (0, 0)),
            pl.BlockSpec((NUM_CLASS,), lambda i: (0,)),
        ],
        out_specs=pl.BlockSpec((rows, NUM_CLASS), lambda i: (i, 0)),
        out_shape=jax.ShapeDtypeStruct((N, NUM_CLASS), jnp.float32),
    )(y2, W2, b2)


def kernel(x, edge_index, edge_weight, W1, b1, Wc1, as1, ad1, bc1, Wc2, as2,
           ad2, bc2, g1, be1, g2, be2, W2, b2):
    src = edge_index[0]
    dst = edge_index[1]
    shp1 = (_NTILE, _NCH, _CH, _K)
    src3a = src.reshape(shp1)
    dst3a = dst.reshape(shp1)
    srcP = src.reshape(4, _NCHP, _CHP, _KP)
    dstP = dst.reshape(4, _NCHP, _CHP, _KP)

    def _padn(a):
        return jnp.pad(a, ((0, _NP - N),) + ((0, 0),) * (a.ndim - 1))

    h0 = _lrelu(x @ W1.T + b1, 0.01)
    h1 = (h0 @ Wc1.T).reshape(N, HEADS, HID)
    a_src = jnp.sum(h1 * as1, axis=-1)
    a_dst = jnp.sum(h1 * ad1, axis=-1)
    aT1 = _padn(a_src).T.reshape(HEADS * _NP)
    bT1 = _padn(a_dst).T.reshape(HEADS * _NP)

    exP1, denP1 = _att(aT1, bT1, srcP, dstP)
    den1 = denP1.reshape(2, 4, 4, _NP).sum(2).reshape(HEADS, _NP)[:, :N].T
    es1 = jnp.exp(_lrelu(a_src + a_dst, 0.2))
    rden1 = 1.0 / (den1 + es1)
    coefT1 = exP1.reshape(HEADS, E).reshape((HEADS,) + shp1)

    hcat = jnp.transpose(
        _padn(h1.reshape(N, 4, 128)), (1, 0, 2)).reshape(4 * _NP, 128)
    selfc = h1 * es1[:, :, None]
    selfcat = jnp.transpose(
        _padn(selfc.reshape(N, 4, 128)), (1, 0, 2)).reshape(4 * _NP, 128)

    out1 = _mp1(hcat, selfcat, src3a, dst3a, coefT1)
    out1 = jnp.transpose(out1.reshape(4, _NP, 128), (1, 0, 2))[:N].reshape(
        N, HEADS, HID)
    out1 = (out1 * rden1[:, :, None]).reshape(N, HEADS * HID)
    y1 = _lrelu(out1 + bc1, 0.01) * (BN_SCALE * g1) + be1

    h2 = y1 @ Wc2.T
    a_src2 = jnp.sum(h2 * as2[0], axis=-1)
    a_dst2 = jnp.sum(h2 * ad2[0], axis=-1)
    z7 = jnp.zeros((7 * _NP,), jnp.float32)
    aT2 = jnp.concatenate([_padn(a_src2), z7])
    bT2 = jnp.concatenate([_padn(a_dst2), z7])

    exP2, denP2 = _att(aT2, bT2, srcP, dstP)
    den2 = (denP2[0:_NP] + denP2[_NP:2 * _NP] + denP2[2 * _NP:3 * _NP]
            + denP2[3 * _NP:4 * _NP])[:N]
    es2 = jnp.exp(_lrelu(a_src2 + a_dst2, 0.2))
    rden2 = 1.0 / (den2 + es2)
    shp2 = (32, 25, 5, _K)
    ex32 = exP2[0].reshape(shp2)
    src32 = src.reshape(shp2)
    dst32 = dst.reshape(shp2)

    h2tab = _padn(h2)
    init2 = jnp.concatenate(
        [_padn(h2 * es2[:, None]), jnp.zeros((_NP, HID2), jnp.float32)],
        axis=0)
    out2p = _mp2n(h2tab, init2, src32, dst32, ex32)
    out2 = (out2p[:_NP] + out2p[_NP:])[:N] * rden2[:, None]
    y2 = _lrelu(out2 + bc2, 0.01) * (BN_SCALE * g2) + be2

    return _tail(y2, W2, b2)
